# baseline (speedup 1.0000x reference)
import functools

import jax
import jax.numpy as jnp
from jax import lax
from jax.experimental import pallas as pl
from jax.experimental.pallas import tpu as pltpu
from jax.experimental.pallas import tpu_sc as plsc

T = 2048
D = 1024
H = 4096
E = 8
K = 2
TBLK = 256
BLK = 256
NBLK = (T * K) // BLK + E
NP = NBLK * BLK
H_BLK = 1024
HB = H // H_BLK
NW = 32
LN_EPS = 1e-5


def _router_body(x_ref, c_ref, wc_ref, wr_ref, rng_ref, rnb_ref, cng_ref,
                 cnb_ref, cvec_ref, brs_ref, idx_ref, gate_ref, imp_ref,
                 z2_ref, ent_ref):
    i = pl.program_id(0)
    xb = x_ref[...]
    cb = c_ref[...]

    def ln(v, g, b):
        m = jnp.mean(v, axis=1, keepdims=True)
        var = jnp.mean((v - m) ** 2, axis=1, keepdims=True)
        return (v - m) * jax.lax.rsqrt(var + LN_EPS) * g + b

    rf = ln(xb, rng_ref[...], rnb_ref[...])
    cn = ln(cb, cng_ref[...], cnb_ref[...])
    rf = rf + lax.dot_general(cn, wc_ref[...], (((1,), (1,)), ((), ())),
                              preferred_element_type=jnp.float32)
    rf = rf + cvec_ref[...]
    logits = lax.dot_general(rf, wr_ref[...], (((1,), (1,)), ((), ())),
                             preferred_element_type=jnp.float32) + brs_ref[...]

    lane = lax.broadcasted_iota(jnp.int32, (TBLK, E), 1)
    m1 = jnp.max(logits, axis=1)
    i1 = jnp.argmax(logits, axis=1).astype(jnp.int32)
    masked = jnp.where(lane == i1[:, None], -jnp.inf, logits)
    m2 = jnp.max(masked, axis=1)
    i2 = jnp.argmax(masked, axis=1).astype(jnp.int32)
    g0 = 1.0 / (1.0 + jnp.exp(m2 - m1))
    g1 = 1.0 - g0
    idx_ref[...] = jnp.where(lane == 0, i1[:, None], i2[:, None])
    gate_ref[...] = jnp.where(lane == 0, g0[:, None], g1[:, None])

    p = jnp.exp(logits - m1[:, None])
    s = jnp.sum(p, axis=1)
    probs = p / s[:, None]
    lse = m1 + jnp.log(s)
    imp_part = jnp.sum(probs, axis=0)[None, :]
    z2_part = jnp.sum(lse * lse)
    ent_part = jnp.sum(probs * jnp.log(jnp.clip(probs, 1e-9, None)))

    @pl.when(i == 0)
    def _():
        imp_ref[...] = imp_part
        z2_ref[...] = jnp.full((1, E), z2_part, jnp.float32)
        ent_ref[...] = jnp.full((1, E), ent_part, jnp.float32)

    @pl.when(i != 0)
    def _():
        imp_ref[...] += imp_part
        z2_ref[...] += jnp.full((1, E), z2_part, jnp.float32)
        ent_ref[...] += jnp.full((1, E), ent_part, jnp.float32)


def _router(x2d, ctx2d, wc, wr_s, rng, rnb, cng, cnb, cvec, br_s):
    grid = (T // TBLK,)
    vrow = pl.BlockSpec((1, D), lambda i: (0, 0))
    out = pl.pallas_call(
        _router_body,
        grid=grid,
        in_specs=[
            pl.BlockSpec((TBLK, D), lambda i: (i, 0)),
            pl.BlockSpec((TBLK, D), lambda i: (i, 0)),
            pl.BlockSpec((D, D), lambda i: (0, 0)),
            pl.BlockSpec((E, D), lambda i: (0, 0)),
            vrow, vrow, vrow, vrow, vrow,
            pl.BlockSpec((1, E), lambda i: (0, 0)),
        ],
        out_specs=[
            pl.BlockSpec((TBLK, E), lambda i: (i, 0)),
            pl.BlockSpec((TBLK, E), lambda i: (i, 0)),
            pl.BlockSpec((1, E), lambda i: (0, 0)),
            pl.BlockSpec((1, E), lambda i: (0, 0)),
            pl.BlockSpec((1, E), lambda i: (0, 0)),
        ],
        out_shape=[
            jax.ShapeDtypeStruct((T, E), jnp.int32),
            jax.ShapeDtypeStruct((T, E), jnp.float32),
            jax.ShapeDtypeStruct((1, E), jnp.float32),
            jax.ShapeDtypeStruct((1, E), jnp.float32),
            jax.ShapeDtypeStruct((1, E), jnp.float32),
        ],
    )(x2d, ctx2d, wc, wr_s, rng, rnb, cng, cnb, cvec, br_s)
    return out


def _erf(z):
    return lax.erf(z)


def _gelu(v):
    return v * 0.5 * (1.0 + _erf(v * 0.7071067811865476))


def _ffn_body(be_ref, nu_ref, xs_ref, w1_ref, b1_ref, w2_ref, b2_ref, g_ref,
              ys_ref, acc_ref):
    b = pl.program_id(0)
    hb = pl.program_id(1)

    @pl.when(b < nu_ref[0])
    def _():
        h = lax.dot_general(xs_ref[...], w1_ref[0],
                            (((1,), (1,)), ((), ())),
                            preferred_element_type=jnp.float32)
        h = _gelu(h + b1_ref[0])
        part = lax.dot_general(h.astype(jnp.bfloat16), w2_ref[0],
                               (((1,), (1,)), ((), ())),
                               preferred_element_type=jnp.float32)

        @pl.when(hb == 0)
        def _():
            acc_ref[...] = part

        @pl.when(hb != 0)
        def _():
            acc_ref[...] += part

        @pl.when(hb == HB - 1)
        def _():
            ys_ref[...] = (acc_ref[...] + b2_ref[0]) * g_ref[...]


def _grouped_ffn(xs, w1, b1, w2, b2, gate_slot, block_expert, nused):
    grid_spec = pltpu.PrefetchScalarGridSpec(
        num_scalar_prefetch=2,
        grid=(NBLK, HB),
        in_specs=[
            pl.BlockSpec((BLK, D), lambda b, hb, be, nu: (b, 0)),
            pl.BlockSpec((1, H_BLK, D), lambda b, hb, be, nu: (be[b], hb, 0)),
            pl.BlockSpec((1, 1, H_BLK), lambda b, hb, be, nu: (be[b], 0, hb)),
            pl.BlockSpec((1, D, H_BLK), lambda b, hb, be, nu: (be[b], 0, hb)),
            pl.BlockSpec((1, 1, D), lambda b, hb, be, nu: (be[b], 0, 0)),
            pl.BlockSpec((BLK, 1), lambda b, hb, be, nu: (b, 0)),
        ],
        out_specs=pl.BlockSpec((BLK, D), lambda b, hb, be, nu: (b, 0)),
        scratch_shapes=[pltpu.VMEM((BLK, D), jnp.float32)],
    )
    return pl.pallas_call(
        _ffn_body,
        grid_spec=grid_spec,
        out_shape=jax.ShapeDtypeStruct((NP, D), jnp.float32),
    )(block_expert, nused, xs, w1, b1, w2, b2, gate_slot)


_GPW = NP // NW
_GCH = 16
_GNC = _GPW // _GCH
_DI = D // 2


def _sc_gather(xi32, rows):
    mesh = plsc.VectorSubcoreMesh(core_axis_name="c", subcore_axis_name="s")

    @functools.partial(
        pl.kernel,
        out_type=jax.ShapeDtypeStruct((NP, _DI), jnp.int32),
        mesh=mesh,
        scratch_types=[
            pltpu.VMEM((_GPW,), jnp.int32),
            pltpu.VMEM((_GCH, _DI), jnp.int32),
            pltpu.VMEM((_GCH, _DI), jnp.int32),
            pltpu.SemaphoreType.DMA,
            pltpu.SemaphoreType.DMA,
            pltpu.SemaphoreType.DMA,
            pltpu.SemaphoreType.DMA,
        ],
    )
    def k(x_hbm, rows_hbm, xs_hbm, idx_v, buf0, buf1, g0, g1, w0, w1):
        wid = lax.axis_index("s") * 2 + lax.axis_index("c")
        base = wid * _GPW
        pltpu.sync_copy(rows_hbm.at[pl.ds(base, _GPW)], idx_v)
        bufs = (buf0, buf1)
        gsems = (g0, g1)
        wsems = (w0, w1)
        gh = [None] * _GNC
        wh = [None] * _GNC
        gh[0] = pltpu.async_copy(
            x_hbm.at[idx_v.at[pl.ds(0, _GCH)]], bufs[0], gsems[0])
        for c in range(_GNC):
            p = c % 2
            gh[c].wait()
            wh[c] = pltpu.async_copy(
                bufs[p], xs_hbm.at[pl.ds(base + c * _GCH, _GCH)], wsems[p])
            if c + 1 < _GNC:
                if c - 1 >= 0:
                    wh[c - 1].wait()
                gh[c + 1] = pltpu.async_copy(
                    x_hbm.at[idx_v.at[pl.ds((c + 1) * _GCH, _GCH)]],
                    bufs[(c + 1) % 2], gsems[(c + 1) % 2])
        wh[_GNC - 2].wait()
        wh[_GNC - 1].wait()

    return k(xi32, rows)


_CCH = 16
_CPW = T // NW


def _sc_combine(ys, inv0, inv1):
    mesh = plsc.VectorSubcoreMesh(core_axis_name="c", subcore_axis_name="s")

    @functools.partial(
        pl.kernel,
        out_type=jax.ShapeDtypeStruct((T, D), jnp.float32),
        mesh=mesh,
        scratch_types=[
            pltpu.VMEM((_CCH,), jnp.int32),
            pltpu.VMEM((_CCH,), jnp.int32),
            pltpu.VMEM((_CCH, D), jnp.float32),
            pltpu.VMEM((_CCH, D), jnp.float32),
            pltpu.SemaphoreType.DMA,
        ],
    )
    def k(ys_hbm, i0_hbm, i1_hbm, out_hbm, i0_v, i1_v, buf0, buf1, sem):
        wid = lax.axis_index("s") * 2 + lax.axis_index("c")
        base = wid * _CPW
        for c in range(_CPW // _CCH):
            off = base + c * _CCH
            pltpu.sync_copy(i0_hbm.at[pl.ds(off, _CCH)], i0_v)
            pltpu.sync_copy(i1_hbm.at[pl.ds(off, _CCH)], i1_v)
            pltpu.async_copy(ys_hbm.at[i0_v], buf0, sem).wait()
            pltpu.async_copy(ys_hbm.at[i1_v], buf1, sem).wait()

            def add_row(r, _):
                for l in range(D // 16):
                    sl = pl.ds(l * 16, 16)
                    buf0[r, sl] = buf0[r, sl] + buf1[r, sl]
                return _

            lax.fori_loop(0, _CCH, add_row, 0)
            pltpu.sync_copy(buf0, out_hbm.at[pl.ds(off, _CCH)])

    return k(ys, inv0, inv1)


def kernel(x, context, quality, params):
    x2d = x.reshape(T, D)
    ctx2d = context.reshape(T, D)

    temp = jnp.clip(params['temp'], 0.25, None)
    wr_s = params['Wr'] / temp
    br_s = (params['br'] / temp).reshape(1, E)
    cvec = (params['bc'] + quality[0] * params['Wq'][:, 0]
            + params['bq']).reshape(1, D)

    idx8, gates8, imp_s, z2_s, ent_s = _router(
        x2d, ctx2d, params['Wc'], wr_s,
        params['rn_g'].reshape(1, D), params['rn_b'].reshape(1, D),
        params['cn_g'].reshape(1, D), params['cn_b'].reshape(1, D),
        cvec, br_s)

    idx2 = idx8[:, :K]
    gates = gates8[:, :K]

    eflat = idx2.reshape(-1)
    counts = jnp.bincount(eflat, length=E)
    blocks_e = (counts + BLK - 1) // BLK
    bstart = jnp.concatenate([jnp.zeros(1, blocks_e.dtype),
                              jnp.cumsum(blocks_e)[:-1]])
    pstart = bstart * BLK
    starts = jnp.concatenate([jnp.zeros(1, counts.dtype),
                              jnp.cumsum(counts)[:-1]])
    order = jnp.argsort(eflat)
    ej = eflat[order]
    j = jnp.arange(T * K)
    ppos = (pstart[ej] + (j - starts[ej])).astype(jnp.int32)
    slot_pair = jnp.zeros(NP, jnp.int32).at[ppos].set(order.astype(jnp.int32))
    valid = jnp.zeros(NP, jnp.float32).at[ppos].set(1.0)
    rows = slot_pair // K
    gate_slot = (valid * gates.reshape(-1)[slot_pair])[:, None]
    barr = jnp.arange(NBLK)
    nused = jnp.sum(blocks_e).astype(jnp.int32)
    bef = (jnp.sum(barr[:, None] >= bstart[None, :], axis=1) - 1)
    bef = jnp.clip(bef, 0, E - 1).astype(jnp.int32)
    block_expert = jnp.where(barr < nused, bef, bef[nused - 1])
    inv = jnp.zeros(T * K, jnp.int32).at[order].set(ppos)
    inv0 = inv[0::K]
    inv1 = inv[1::K]

    x16 = x2d.astype(jnp.bfloat16)
    xi32 = lax.bitcast_convert_type(x16.reshape(T, _DI, 2), jnp.int32)
    xs_i = _sc_gather(xi32, rows)
    xs = lax.bitcast_convert_type(xs_i, jnp.bfloat16).reshape(NP, D)
    ys = _grouped_ffn(xs, params['W1'].astype(jnp.bfloat16),
                      params['b1'].reshape(E, 1, H),
                      params['W2'].astype(jnp.bfloat16),
                      params['b2'].reshape(E, 1, D),
                      gate_slot, block_expert, nused.reshape(1))
    out2d = _sc_combine(ys, inv0, inv1)

    invT = 1.0 / jnp.float32(T)
    importance = imp_s[0] * invT
    load_balance = jnp.mean((importance - 1.0 / E) ** 2)
    router_z = z2_s[0, 0] * invT
    entropy = -ent_s[0, 0] * invT
    aux = load_balance + 0.001 * router_z - 0.001 * entropy
    return (out2d.reshape(1, T, D), load_balance, router_z, entropy, aux)


# speedup vs baseline: 1.5077x; 1.5077x over previous
import functools

import jax
import jax.numpy as jnp
from jax import lax
from jax.experimental import pallas as pl
from jax.experimental.pallas import tpu as pltpu
from jax.experimental.pallas import tpu_sc as plsc

T = 2048
D = 1024
H = 4096
E = 8
K = 2
TBLK = 256
BLK = 256
NBLK = (T * K) // BLK + E
NP = NBLK * BLK
H_BLK = 1024
HB = H // H_BLK
NW = 32
LN_EPS = 1e-5


def _router_body(x_ref, c_ref, wc_ref, wr_ref, rng_ref, rnb_ref, cng_ref,
                 cnb_ref, cvec_ref, brs_ref, idx_ref, gate_ref, imp_ref,
                 z2_ref, ent_ref):
    i = pl.program_id(0)
    xb = x_ref[...]
    cb = c_ref[...]

    def ln(v, g, b):
        m = jnp.mean(v, axis=1, keepdims=True)
        var = jnp.mean((v - m) ** 2, axis=1, keepdims=True)
        return (v - m) * jax.lax.rsqrt(var + LN_EPS) * g + b

    rf = ln(xb, rng_ref[...], rnb_ref[...])
    cn = ln(cb, cng_ref[...], cnb_ref[...])
    rf = rf + lax.dot_general(cn, wc_ref[...], (((1,), (1,)), ((), ())),
                              preferred_element_type=jnp.float32)
    rf = rf + cvec_ref[...]
    logits = lax.dot_general(rf, wr_ref[...], (((1,), (1,)), ((), ())),
                             preferred_element_type=jnp.float32) + brs_ref[...]

    lane = lax.broadcasted_iota(jnp.int32, (TBLK, E), 1)
    m1 = jnp.max(logits, axis=1)
    i1 = jnp.argmax(logits, axis=1).astype(jnp.int32)
    masked = jnp.where(lane == i1[:, None], -jnp.inf, logits)
    m2 = jnp.max(masked, axis=1)
    i2 = jnp.argmax(masked, axis=1).astype(jnp.int32)
    g0 = 1.0 / (1.0 + jnp.exp(m2 - m1))
    g1 = 1.0 - g0
    idx_ref[...] = jnp.where(lane == 0, i1[:, None], i2[:, None])
    gate_ref[...] = jnp.where(lane == 0, g0[:, None], g1[:, None])

    p = jnp.exp(logits - m1[:, None])
    s = jnp.sum(p, axis=1)
    probs = p / s[:, None]
    lse = m1 + jnp.log(s)
    imp_part = jnp.sum(probs, axis=0)[None, :]
    z2_part = jnp.sum(lse * lse)
    ent_part = jnp.sum(probs * jnp.log(jnp.clip(probs, 1e-9, None)))

    @pl.when(i == 0)
    def _():
        imp_ref[...] = imp_part
        z2_ref[...] = jnp.full((1, E), z2_part, jnp.float32)
        ent_ref[...] = jnp.full((1, E), ent_part, jnp.float32)

    @pl.when(i != 0)
    def _():
        imp_ref[...] += imp_part
        z2_ref[...] += jnp.full((1, E), z2_part, jnp.float32)
        ent_ref[...] += jnp.full((1, E), ent_part, jnp.float32)


def _router(x2d, ctx2d, wc, wr_s, rng, rnb, cng, cnb, cvec, br_s):
    grid = (T // TBLK,)
    vrow = pl.BlockSpec((1, D), lambda i: (0, 0))
    out = pl.pallas_call(
        _router_body,
        grid=grid,
        in_specs=[
            pl.BlockSpec((TBLK, D), lambda i: (i, 0)),
            pl.BlockSpec((TBLK, D), lambda i: (i, 0)),
            pl.BlockSpec((D, D), lambda i: (0, 0)),
            pl.BlockSpec((E, D), lambda i: (0, 0)),
            vrow, vrow, vrow, vrow, vrow,
            pl.BlockSpec((1, E), lambda i: (0, 0)),
        ],
        out_specs=[
            pl.BlockSpec((TBLK, E), lambda i: (i, 0)),
            pl.BlockSpec((TBLK, E), lambda i: (i, 0)),
            pl.BlockSpec((1, E), lambda i: (0, 0)),
            pl.BlockSpec((1, E), lambda i: (0, 0)),
            pl.BlockSpec((1, E), lambda i: (0, 0)),
        ],
        out_shape=[
            jax.ShapeDtypeStruct((T, E), jnp.int32),
            jax.ShapeDtypeStruct((T, E), jnp.float32),
            jax.ShapeDtypeStruct((1, E), jnp.float32),
            jax.ShapeDtypeStruct((1, E), jnp.float32),
            jax.ShapeDtypeStruct((1, E), jnp.float32),
        ],
    )(x2d, ctx2d, wc, wr_s, rng, rnb, cng, cnb, cvec, br_s)
    return out


def _erf(z):
    return lax.erf(z)


def _gelu(v):
    return v * 0.5 * (1.0 + _erf(v * 0.7071067811865476))


def _ffn_body(be_ref, nu_ref, xs_ref, w1_ref, b1_ref, w2_ref, b2_ref, g_ref,
              ys_ref, acc_ref):
    b = pl.program_id(0)
    hb = pl.program_id(1)

    @pl.when(b < nu_ref[0])
    def _():
        h = lax.dot_general(xs_ref[...].astype(jnp.bfloat16), w1_ref[0],
                            (((1,), (1,)), ((), ())),
                            preferred_element_type=jnp.float32)
        h = _gelu(h + b1_ref[0])
        part = lax.dot_general(h.astype(jnp.bfloat16), w2_ref[0],
                               (((1,), (1,)), ((), ())),
                               preferred_element_type=jnp.float32)

        @pl.when(hb == 0)
        def _():
            acc_ref[...] = part

        @pl.when(hb != 0)
        def _():
            acc_ref[...] += part

        @pl.when(hb == HB - 1)
        def _():
            ys_ref[...] = (acc_ref[...] + b2_ref[0]) * g_ref[...]


def _grouped_ffn(xs, w1, b1, w2, b2, gate_slot, block_expert, nused):
    grid_spec = pltpu.PrefetchScalarGridSpec(
        num_scalar_prefetch=2,
        grid=(NBLK, HB),
        in_specs=[
            pl.BlockSpec((BLK, D), lambda b, hb, be, nu: (b, 0)),
            pl.BlockSpec((1, H_BLK, D), lambda b, hb, be, nu: (be[b], hb, 0)),
            pl.BlockSpec((1, 1, H_BLK), lambda b, hb, be, nu: (be[b], 0, hb)),
            pl.BlockSpec((1, D, H_BLK), lambda b, hb, be, nu: (be[b], 0, hb)),
            pl.BlockSpec((1, 1, D), lambda b, hb, be, nu: (be[b], 0, 0)),
            pl.BlockSpec((BLK, 1), lambda b, hb, be, nu: (b, 0)),
        ],
        out_specs=pl.BlockSpec((BLK, D), lambda b, hb, be, nu: (b, 0)),
        scratch_shapes=[pltpu.VMEM((BLK, D), jnp.float32)],
    )
    return pl.pallas_call(
        _ffn_body,
        grid_spec=grid_spec,
        out_shape=jax.ShapeDtypeStruct((NP, D), jnp.float32),
    )(block_expert, nused, xs, w1, b1, w2, b2, gate_slot)


_GPW = NP // NW
_GCH = 16
_GNC = _GPW // _GCH


def _sc_gather(x2d, rows):
    mesh = plsc.VectorSubcoreMesh(core_axis_name="c", subcore_axis_name="s")

    @functools.partial(
        pl.kernel,
        out_type=jax.ShapeDtypeStruct((NP, D), jnp.float32),
        mesh=mesh,
        scratch_types=[
            pltpu.VMEM((_GPW,), jnp.int32),
            pltpu.VMEM((_GCH, D), jnp.float32),
            pltpu.VMEM((_GCH, D), jnp.float32),
            pltpu.SemaphoreType.DMA,
            pltpu.SemaphoreType.DMA,
            pltpu.SemaphoreType.DMA,
            pltpu.SemaphoreType.DMA,
        ],
    )
    def k(x_hbm, rows_hbm, xs_hbm, idx_v, buf0, buf1, g0, g1, w0, w1):
        wid = lax.axis_index("s") * 2 + lax.axis_index("c")
        base = wid * _GPW
        pltpu.sync_copy(rows_hbm.at[pl.ds(base, _GPW)], idx_v)
        bufs = (buf0, buf1)
        gsems = (g0, g1)
        wsems = (w0, w1)
        gh = [None] * _GNC
        wh = [None] * _GNC
        gh[0] = pltpu.async_copy(
            x_hbm.at[idx_v.at[pl.ds(0, _GCH)]], bufs[0], gsems[0])
        for c in range(_GNC):
            p = c % 2
            gh[c].wait()
            wh[c] = pltpu.async_copy(
                bufs[p], xs_hbm.at[pl.ds(base + c * _GCH, _GCH)], wsems[p])
            if c + 1 < _GNC:
                if c - 1 >= 0:
                    wh[c - 1].wait()
                gh[c + 1] = pltpu.async_copy(
                    x_hbm.at[idx_v.at[pl.ds((c + 1) * _GCH, _GCH)]],
                    bufs[(c + 1) % 2], gsems[(c + 1) % 2])
        wh[_GNC - 2].wait()
        wh[_GNC - 1].wait()

    return k(x2d, rows)


_CCH = 16
_CPW = T // NW


def _sc_combine(ys, inv0, inv1):
    mesh = plsc.VectorSubcoreMesh(core_axis_name="c", subcore_axis_name="s")

    @functools.partial(
        pl.kernel,
        out_type=jax.ShapeDtypeStruct((T, D), jnp.float32),
        mesh=mesh,
        scratch_types=[
            pltpu.VMEM((_CCH,), jnp.int32),
            pltpu.VMEM((_CCH,), jnp.int32),
            pltpu.VMEM((_CCH, D), jnp.float32),
            pltpu.VMEM((_CCH, D), jnp.float32),
            pltpu.SemaphoreType.DMA,
        ],
    )
    def k(ys_hbm, i0_hbm, i1_hbm, out_hbm, i0_v, i1_v, buf0, buf1, sem):
        wid = lax.axis_index("s") * 2 + lax.axis_index("c")
        base = wid * _CPW
        for c in range(_CPW // _CCH):
            off = base + c * _CCH
            pltpu.sync_copy(i0_hbm.at[pl.ds(off, _CCH)], i0_v)
            pltpu.sync_copy(i1_hbm.at[pl.ds(off, _CCH)], i1_v)
            pltpu.async_copy(ys_hbm.at[i0_v], buf0, sem).wait()
            pltpu.async_copy(ys_hbm.at[i1_v], buf1, sem).wait()

            def add_row(r, _):
                for l in range(D // 16):
                    sl = pl.ds(l * 16, 16)
                    buf0[r, sl] = buf0[r, sl] + buf1[r, sl]
                return _

            lax.fori_loop(0, _CCH, add_row, 0)
            pltpu.sync_copy(buf0, out_hbm.at[pl.ds(off, _CCH)])

    return k(ys, inv0, inv1)


def kernel(x, context, quality, params):
    x2d = x.reshape(T, D)
    ctx2d = context.reshape(T, D)

    temp = jnp.clip(params['temp'], 0.25, None)
    wr_s = params['Wr'] / temp
    br_s = (params['br'] / temp).reshape(1, E)
    cvec = (params['bc'] + quality[0] * params['Wq'][:, 0]
            + params['bq']).reshape(1, D)

    idx8, gates8, imp_s, z2_s, ent_s = _router(
        x2d, ctx2d, params['Wc'], wr_s,
        params['rn_g'].reshape(1, D), params['rn_b'].reshape(1, D),
        params['cn_g'].reshape(1, D), params['cn_b'].reshape(1, D),
        cvec, br_s)

    idx2 = idx8[:, :K]
    gates = gates8[:, :K]

    eflat = idx2.reshape(-1)
    counts = jnp.bincount(eflat, length=E)
    blocks_e = (counts + BLK - 1) // BLK
    bstart = jnp.concatenate([jnp.zeros(1, blocks_e.dtype),
                              jnp.cumsum(blocks_e)[:-1]])
    pstart = bstart * BLK
    starts = jnp.concatenate([jnp.zeros(1, counts.dtype),
                              jnp.cumsum(counts)[:-1]])
    order = jnp.argsort(eflat)
    ej = eflat[order]
    j = jnp.arange(T * K)
    ppos = (pstart[ej] + (j - starts[ej])).astype(jnp.int32)
    slot_pair = (jnp.arange(NP, dtype=jnp.int32) % (T * K)).at[ppos].set(
        order.astype(jnp.int32))
    valid = jnp.zeros(NP, jnp.float32).at[ppos].set(1.0)
    rows = slot_pair // K
    gate_slot = (valid * gates.reshape(-1)[slot_pair])[:, None]
    barr = jnp.arange(NBLK)
    nused = jnp.sum(blocks_e).astype(jnp.int32)
    bef = (jnp.sum(barr[:, None] >= bstart[None, :], axis=1) - 1)
    bef = jnp.clip(bef, 0, E - 1).astype(jnp.int32)
    block_expert = jnp.where(barr < nused, bef, bef[nused - 1])
    inv = jnp.zeros(T * K, jnp.int32).at[order].set(ppos)
    inv0 = inv[0::K]
    inv1 = inv[1::K]

    xs = _sc_gather(x2d, rows)
    ys = _grouped_ffn(xs, params['W1'].astype(jnp.bfloat16),
                      params['b1'].reshape(E, 1, H),
                      params['W2'].astype(jnp.bfloat16),
                      params['b2'].reshape(E, 1, D),
                      gate_slot, block_expert, nused.reshape(1))
    out2d = _sc_combine(ys, inv0, inv1)

    invT = 1.0 / jnp.float32(T)
    importance = imp_s[0] * invT
    load_balance = jnp.mean((importance - 1.0 / E) ** 2)
    router_z = z2_s[0, 0] * invT
    entropy = -ent_s[0, 0] * invT
    aux = load_balance + 0.001 * router_z - 0.001 * entropy
    return (out2d.reshape(1, T, D), load_balance, router_z, entropy, aux)


# speedup vs baseline: 1.6262x; 1.0786x over previous
import functools

import jax
import jax.numpy as jnp
from jax import lax
from jax.experimental import pallas as pl
from jax.experimental.pallas import tpu as pltpu
from jax.experimental.pallas import tpu_sc as plsc

T = 2048
D = 1024
H = 4096
E = 8
K = 2
TBLK = 256
BLK = 256
NBLK = (T * K) // BLK + E
NP = NBLK * BLK
H_BLK = 2048
HB = H // H_BLK
NW = 32
LN_EPS = 1e-5


def _router_body(x_ref, c_ref, wc_ref, wr_ref, rng_ref, rnb_ref, cng_ref,
                 cnb_ref, cvec_ref, brs_ref, idx_ref, gate_ref, imp_ref,
                 z2_ref, ent_ref):
    i = pl.program_id(0)
    xb = x_ref[...]
    cb = c_ref[...]

    def ln(v, g, b):
        m = jnp.mean(v, axis=1, keepdims=True)
        var = jnp.mean((v - m) ** 2, axis=1, keepdims=True)
        return (v - m) * jax.lax.rsqrt(var + LN_EPS) * g + b

    rf = ln(xb, rng_ref[...], rnb_ref[...])
    cn = ln(cb, cng_ref[...], cnb_ref[...])
    rf = rf + lax.dot_general(cn, wc_ref[...], (((1,), (1,)), ((), ())),
                              preferred_element_type=jnp.float32)
    rf = rf + cvec_ref[...]
    logits = lax.dot_general(rf, wr_ref[...], (((1,), (1,)), ((), ())),
                             preferred_element_type=jnp.float32) + brs_ref[...]

    lane = lax.broadcasted_iota(jnp.int32, (TBLK, E), 1)
    m1 = jnp.max(logits, axis=1)
    i1 = jnp.argmax(logits, axis=1).astype(jnp.int32)
    masked = jnp.where(lane == i1[:, None], -jnp.inf, logits)
    m2 = jnp.max(masked, axis=1)
    i2 = jnp.argmax(masked, axis=1).astype(jnp.int32)
    g0 = 1.0 / (1.0 + jnp.exp(m2 - m1))
    g1 = 1.0 - g0
    idx_ref[...] = jnp.where(lane == 0, i1[:, None], i2[:, None])
    gate_ref[...] = jnp.where(lane == 0, g0[:, None], g1[:, None])

    p = jnp.exp(logits - m1[:, None])
    s = jnp.sum(p, axis=1)
    probs = p / s[:, None]
    lse = m1 + jnp.log(s)
    imp_part = jnp.sum(probs, axis=0)[None, :]
    z2_part = jnp.sum(lse * lse)
    ent_part = jnp.sum(probs * jnp.log(jnp.clip(probs, 1e-9, None)))

    @pl.when(i == 0)
    def _():
        imp_ref[...] = imp_part
        z2_ref[...] = jnp.full((1, E), z2_part, jnp.float32)
        ent_ref[...] = jnp.full((1, E), ent_part, jnp.float32)

    @pl.when(i != 0)
    def _():
        imp_ref[...] += imp_part
        z2_ref[...] += jnp.full((1, E), z2_part, jnp.float32)
        ent_ref[...] += jnp.full((1, E), ent_part, jnp.float32)


def _router(x2d, ctx2d, wc, wr_s, rng, rnb, cng, cnb, cvec, br_s):
    grid = (T // TBLK,)
    vrow = pl.BlockSpec((1, D), lambda i: (0, 0))
    out = pl.pallas_call(
        _router_body,
        grid=grid,
        in_specs=[
            pl.BlockSpec((TBLK, D), lambda i: (i, 0)),
            pl.BlockSpec((TBLK, D), lambda i: (i, 0)),
            pl.BlockSpec((D, D), lambda i: (0, 0)),
            pl.BlockSpec((E, D), lambda i: (0, 0)),
            vrow, vrow, vrow, vrow, vrow,
            pl.BlockSpec((1, E), lambda i: (0, 0)),
        ],
        out_specs=[
            pl.BlockSpec((TBLK, E), lambda i: (i, 0)),
            pl.BlockSpec((TBLK, E), lambda i: (i, 0)),
            pl.BlockSpec((1, E), lambda i: (0, 0)),
            pl.BlockSpec((1, E), lambda i: (0, 0)),
            pl.BlockSpec((1, E), lambda i: (0, 0)),
        ],
        out_shape=[
            jax.ShapeDtypeStruct((T, E), jnp.int32),
            jax.ShapeDtypeStruct((T, E), jnp.float32),
            jax.ShapeDtypeStruct((1, E), jnp.float32),
            jax.ShapeDtypeStruct((1, E), jnp.float32),
            jax.ShapeDtypeStruct((1, E), jnp.float32),
        ],
    )(x2d, ctx2d, wc, wr_s, rng, rnb, cng, cnb, cvec, br_s)
    return out


def _erf(z):
    return lax.erf(z)


def _gelu(v):
    return v * 0.5 * (1.0 + _erf(v * 0.7071067811865476))


def _ffn_body(be_ref, nu_ref, xs_ref, w1_ref, b1_ref, w2_ref, b2_ref, g_ref,
              ys_ref, acc_ref):
    b = pl.program_id(0)
    hb = pl.program_id(1)

    @pl.when(b < nu_ref[0])
    def _():
        h = lax.dot_general(xs_ref[...].astype(jnp.bfloat16), w1_ref[0],
                            (((1,), (1,)), ((), ())),
                            preferred_element_type=jnp.float32)
        h = _gelu(h + b1_ref[0])
        part = lax.dot_general(h.astype(jnp.bfloat16), w2_ref[0],
                               (((1,), (1,)), ((), ())),
                               preferred_element_type=jnp.float32)

        @pl.when(hb == 0)
        def _():
            acc_ref[...] = part

        @pl.when(hb != 0)
        def _():
            acc_ref[...] += part

        @pl.when(hb == HB - 1)
        def _():
            ys_ref[...] = (acc_ref[...] + b2_ref[0]) * g_ref[...]


def _grouped_ffn(xs, w1, b1, w2, b2, gate_slot, block_expert, nused):
    grid_spec = pltpu.PrefetchScalarGridSpec(
        num_scalar_prefetch=2,
        grid=(NBLK, HB),
        in_specs=[
            pl.BlockSpec((BLK, D), lambda b, hb, be, nu: (b, 0)),
            pl.BlockSpec((1, H_BLK, D), lambda b, hb, be, nu: (be[b], hb, 0)),
            pl.BlockSpec((1, 1, H_BLK), lambda b, hb, be, nu: (be[b], 0, hb)),
            pl.BlockSpec((1, D, H_BLK), lambda b, hb, be, nu: (be[b], 0, hb)),
            pl.BlockSpec((1, 1, D), lambda b, hb, be, nu: (be[b], 0, 0)),
            pl.BlockSpec((BLK, 1), lambda b, hb, be, nu: (b, 0)),
        ],
        out_specs=pl.BlockSpec((BLK, D), lambda b, hb, be, nu: (b, 0)),
        scratch_shapes=[pltpu.VMEM((BLK, D), jnp.float32)],
    )
    return pl.pallas_call(
        _ffn_body,
        grid_spec=grid_spec,
        out_shape=jax.ShapeDtypeStruct((NP, D), jnp.float32),
    )(block_expert, nused, xs, w1, b1, w2, b2, gate_slot)


_GPW = NP // NW
_GCH = 16
_GNC = _GPW // _GCH


def _sc_gather(x2d, rows):
    mesh = plsc.VectorSubcoreMesh(core_axis_name="c", subcore_axis_name="s")

    @functools.partial(
        pl.kernel,
        out_type=jax.ShapeDtypeStruct((NP, D), jnp.float32),
        mesh=mesh,
        scratch_types=[
            pltpu.VMEM((_GPW,), jnp.int32),
            pltpu.VMEM((_GCH, D), jnp.float32),
            pltpu.VMEM((_GCH, D), jnp.float32),
            pltpu.SemaphoreType.DMA,
            pltpu.SemaphoreType.DMA,
            pltpu.SemaphoreType.DMA,
            pltpu.SemaphoreType.DMA,
        ],
    )
    def k(x_hbm, rows_hbm, xs_hbm, idx_v, buf0, buf1, g0, g1, w0, w1):
        wid = lax.axis_index("s") * 2 + lax.axis_index("c")
        base = wid * _GPW
        pltpu.sync_copy(rows_hbm.at[pl.ds(base, _GPW)], idx_v)
        bufs = (buf0, buf1)
        gsems = (g0, g1)
        wsems = (w0, w1)
        gh = [None] * _GNC
        wh = [None] * _GNC
        gh[0] = pltpu.async_copy(
            x_hbm.at[idx_v.at[pl.ds(0, _GCH)]], bufs[0], gsems[0])
        for c in range(_GNC):
            p = c % 2
            gh[c].wait()
            wh[c] = pltpu.async_copy(
                bufs[p], xs_hbm.at[pl.ds(base + c * _GCH, _GCH)], wsems[p])
            if c + 1 < _GNC:
                if c - 1 >= 0:
                    wh[c - 1].wait()
                gh[c + 1] = pltpu.async_copy(
                    x_hbm.at[idx_v.at[pl.ds((c + 1) * _GCH, _GCH)]],
                    bufs[(c + 1) % 2], gsems[(c + 1) % 2])
        wh[_GNC - 2].wait()
        wh[_GNC - 1].wait()

    return k(x2d, rows)


_CCH = 16
_CPW = T // NW


def _sc_combine(ys, inv0, inv1):
    mesh = plsc.VectorSubcoreMesh(core_axis_name="c", subcore_axis_name="s")

    @functools.partial(
        pl.kernel,
        out_type=jax.ShapeDtypeStruct((T, D), jnp.float32),
        mesh=mesh,
        scratch_types=[
            pltpu.VMEM((_CCH,), jnp.int32),
            pltpu.VMEM((_CCH,), jnp.int32),
            pltpu.VMEM((_CCH, D), jnp.float32),
            pltpu.VMEM((_CCH, D), jnp.float32),
            pltpu.SemaphoreType.DMA,
        ],
    )
    def k(ys_hbm, i0_hbm, i1_hbm, out_hbm, i0_v, i1_v, buf0, buf1, sem):
        wid = lax.axis_index("s") * 2 + lax.axis_index("c")
        base = wid * _CPW
        for c in range(_CPW // _CCH):
            off = base + c * _CCH
            pltpu.sync_copy(i0_hbm.at[pl.ds(off, _CCH)], i0_v)
            pltpu.sync_copy(i1_hbm.at[pl.ds(off, _CCH)], i1_v)
            pltpu.async_copy(ys_hbm.at[i0_v], buf0, sem).wait()
            pltpu.async_copy(ys_hbm.at[i1_v], buf1, sem).wait()

            def add_row(r, _):
                for l in range(D // 16):
                    sl = pl.ds(l * 16, 16)
                    buf0[r, sl] = buf0[r, sl] + buf1[r, sl]
                return _

            lax.fori_loop(0, _CCH, add_row, 0)
            pltpu.sync_copy(buf0, out_hbm.at[pl.ds(off, _CCH)])

    return k(ys, inv0, inv1)


def kernel(x, context, quality, params):
    x2d = x.reshape(T, D)
    ctx2d = context.reshape(T, D)

    temp = jnp.clip(params['temp'], 0.25, None)
    wr_s = params['Wr'] / temp
    br_s = (params['br'] / temp).reshape(1, E)
    cvec = (params['bc'] + quality[0] * params['Wq'][:, 0]
            + params['bq']).reshape(1, D)

    idx8, gates8, imp_s, z2_s, ent_s = _router(
        x2d, ctx2d, params['Wc'], wr_s,
        params['rn_g'].reshape(1, D), params['rn_b'].reshape(1, D),
        params['cn_g'].reshape(1, D), params['cn_b'].reshape(1, D),
        cvec, br_s)

    idx2 = idx8[:, :K]
    gates = gates8[:, :K]

    eflat = idx2.reshape(-1)
    counts = jnp.bincount(eflat, length=E)
    blocks_e = (counts + BLK - 1) // BLK
    bstart = jnp.concatenate([jnp.zeros(1, blocks_e.dtype),
                              jnp.cumsum(blocks_e)[:-1]])
    pstart = bstart * BLK
    starts = jnp.concatenate([jnp.zeros(1, counts.dtype),
                              jnp.cumsum(counts)[:-1]])
    order = jnp.argsort(eflat)
    ej = eflat[order]
    j = jnp.arange(T * K)
    ppos = (pstart[ej] + (j - starts[ej])).astype(jnp.int32)
    slot_pair = (jnp.arange(NP, dtype=jnp.int32) % (T * K)).at[ppos].set(
        order.astype(jnp.int32))
    valid = jnp.zeros(NP, jnp.float32).at[ppos].set(1.0)
    rows = slot_pair // K
    gate_slot = (valid * gates.reshape(-1)[slot_pair])[:, None]
    barr = jnp.arange(NBLK)
    nused = jnp.sum(blocks_e).astype(jnp.int32)
    bef = (jnp.sum(barr[:, None] >= bstart[None, :], axis=1) - 1)
    bef = jnp.clip(bef, 0, E - 1).astype(jnp.int32)
    block_expert = jnp.where(barr < nused, bef, bef[nused - 1])
    inv = jnp.zeros(T * K, jnp.int32).at[order].set(ppos)
    inv0 = inv[0::K]
    inv1 = inv[1::K]

    xs = _sc_gather(x2d, rows)
    ys = _grouped_ffn(xs, params['W1'].astype(jnp.bfloat16),
                      params['b1'].reshape(E, 1, H),
                      params['W2'].astype(jnp.bfloat16),
                      params['b2'].reshape(E, 1, D),
                      gate_slot, block_expert, nused.reshape(1))
    out2d = _sc_combine(ys, inv0, inv1)

    invT = 1.0 / jnp.float32(T)
    importance = imp_s[0] * invT
    load_balance = jnp.mean((importance - 1.0 / E) ** 2)
    router_z = z2_s[0, 0] * invT
    entropy = -ent_s[0, 0] * invT
    aux = load_balance + 0.001 * router_z - 0.001 * entropy
    return (out2d.reshape(1, T, D), load_balance, router_z, entropy, aux)


# speedup vs baseline: 1.7503x; 1.0763x over previous
import functools

import jax
import jax.numpy as jnp
from jax import lax
from jax.experimental import pallas as pl
from jax.experimental.pallas import tpu as pltpu
from jax.experimental.pallas import tpu_sc as plsc

T = 2048
D = 1024
H = 4096
E = 8
K = 2
TBLK = 256
BLK = 256
NBLK = (T * K) // BLK + E
NP = NBLK * BLK
H_BLK = 4096
HB = H // H_BLK
NW = 32
LN_EPS = 1e-5


def _router_body(x_ref, c_ref, wc_ref, wr_ref, rng_ref, rnb_ref, cng_ref,
                 cnb_ref, cvec_ref, brs_ref, idx_ref, gate_ref, imp_ref,
                 z2_ref, ent_ref):
    i = pl.program_id(0)
    xb = x_ref[...]
    cb = c_ref[...]

    def ln(v, g, b):
        m = jnp.mean(v, axis=1, keepdims=True)
        var = jnp.mean((v - m) ** 2, axis=1, keepdims=True)
        return (v - m) * jax.lax.rsqrt(var + LN_EPS) * g + b

    rf = ln(xb, rng_ref[...], rnb_ref[...])
    cn = ln(cb, cng_ref[...], cnb_ref[...])
    rf = rf + lax.dot_general(cn, wc_ref[...], (((1,), (1,)), ((), ())),
                              preferred_element_type=jnp.float32)
    rf = rf + cvec_ref[...]
    logits = lax.dot_general(rf, wr_ref[...], (((1,), (1,)), ((), ())),
                             preferred_element_type=jnp.float32) + brs_ref[...]

    lane = lax.broadcasted_iota(jnp.int32, (TBLK, E), 1)
    m1 = jnp.max(logits, axis=1)
    i1 = jnp.argmax(logits, axis=1).astype(jnp.int32)
    masked = jnp.where(lane == i1[:, None], -jnp.inf, logits)
    m2 = jnp.max(masked, axis=1)
    i2 = jnp.argmax(masked, axis=1).astype(jnp.int32)
    g0 = 1.0 / (1.0 + jnp.exp(m2 - m1))
    g1 = 1.0 - g0
    idx_ref[...] = jnp.where(lane == 0, i1[:, None], i2[:, None])
    gate_ref[...] = jnp.where(lane == 0, g0[:, None], g1[:, None])

    p = jnp.exp(logits - m1[:, None])
    s = jnp.sum(p, axis=1)
    probs = p / s[:, None]
    lse = m1 + jnp.log(s)
    imp_part = jnp.sum(probs, axis=0)[None, :]
    z2_part = jnp.sum(lse * lse)
    ent_part = jnp.sum(probs * jnp.log(jnp.clip(probs, 1e-9, None)))

    @pl.when(i == 0)
    def _():
        imp_ref[...] = imp_part
        z2_ref[...] = jnp.full((1, E), z2_part, jnp.float32)
        ent_ref[...] = jnp.full((1, E), ent_part, jnp.float32)

    @pl.when(i != 0)
    def _():
        imp_ref[...] += imp_part
        z2_ref[...] += jnp.full((1, E), z2_part, jnp.float32)
        ent_ref[...] += jnp.full((1, E), ent_part, jnp.float32)


def _router(x2d, ctx2d, wc, wr_s, rng, rnb, cng, cnb, cvec, br_s):
    grid = (T // TBLK,)
    vrow = pl.BlockSpec((1, D), lambda i: (0, 0))
    out = pl.pallas_call(
        _router_body,
        grid=grid,
        in_specs=[
            pl.BlockSpec((TBLK, D), lambda i: (i, 0)),
            pl.BlockSpec((TBLK, D), lambda i: (i, 0)),
            pl.BlockSpec((D, D), lambda i: (0, 0)),
            pl.BlockSpec((E, D), lambda i: (0, 0)),
            vrow, vrow, vrow, vrow, vrow,
            pl.BlockSpec((1, E), lambda i: (0, 0)),
        ],
        out_specs=[
            pl.BlockSpec((TBLK, E), lambda i: (i, 0)),
            pl.BlockSpec((TBLK, E), lambda i: (i, 0)),
            pl.BlockSpec((1, E), lambda i: (0, 0)),
            pl.BlockSpec((1, E), lambda i: (0, 0)),
            pl.BlockSpec((1, E), lambda i: (0, 0)),
        ],
        out_shape=[
            jax.ShapeDtypeStruct((T, E), jnp.int32),
            jax.ShapeDtypeStruct((T, E), jnp.float32),
            jax.ShapeDtypeStruct((1, E), jnp.float32),
            jax.ShapeDtypeStruct((1, E), jnp.float32),
            jax.ShapeDtypeStruct((1, E), jnp.float32),
        ],
    )(x2d, ctx2d, wc, wr_s, rng, rnb, cng, cnb, cvec, br_s)
    return out


def _erf(z):
    return lax.erf(z)


def _gelu(v):
    return v * 0.5 * (1.0 + _erf(v * 0.7071067811865476))


def _ffn_body(be_ref, nu_ref, xs_ref, w1_ref, b1_ref, w2_ref, b2_ref, g_ref,
              ys_ref, acc_ref):
    b = pl.program_id(0)
    hb = pl.program_id(1)

    @pl.when(b < nu_ref[0])
    def _():
        h = lax.dot_general(xs_ref[...].astype(jnp.bfloat16), w1_ref[0],
                            (((1,), (1,)), ((), ())),
                            preferred_element_type=jnp.float32)
        h = _gelu(h + b1_ref[0])
        part = lax.dot_general(h.astype(jnp.bfloat16), w2_ref[0],
                               (((1,), (1,)), ((), ())),
                               preferred_element_type=jnp.float32)

        @pl.when(hb == 0)
        def _():
            acc_ref[...] = part

        @pl.when(hb != 0)
        def _():
            acc_ref[...] += part

        @pl.when(hb == HB - 1)
        def _():
            ys_ref[...] = (acc_ref[...] + b2_ref[0]) * g_ref[...]


def _grouped_ffn(xs, w1, b1, w2, b2, gate_slot, block_expert, nused):
    grid_spec = pltpu.PrefetchScalarGridSpec(
        num_scalar_prefetch=2,
        grid=(NBLK, HB),
        in_specs=[
            pl.BlockSpec((BLK, D), lambda b, hb, be, nu: (b, 0)),
            pl.BlockSpec((1, H_BLK, D), lambda b, hb, be, nu: (be[b], hb, 0)),
            pl.BlockSpec((1, 1, H_BLK), lambda b, hb, be, nu: (be[b], 0, hb)),
            pl.BlockSpec((1, D, H_BLK), lambda b, hb, be, nu: (be[b], 0, hb)),
            pl.BlockSpec((1, 1, D), lambda b, hb, be, nu: (be[b], 0, 0)),
            pl.BlockSpec((BLK, 1), lambda b, hb, be, nu: (b, 0)),
        ],
        out_specs=pl.BlockSpec((BLK, D), lambda b, hb, be, nu: (b, 0)),
        scratch_shapes=[pltpu.VMEM((BLK, D), jnp.float32)],
    )
    return pl.pallas_call(
        _ffn_body,
        grid_spec=grid_spec,
        out_shape=jax.ShapeDtypeStruct((NP, D), jnp.float32),
    )(block_expert, nused, xs, w1, b1, w2, b2, gate_slot)


_GPW = NP // NW
_GCH = 16
_GNC = _GPW // _GCH


def _sc_gather(x2d, rows):
    mesh = plsc.VectorSubcoreMesh(core_axis_name="c", subcore_axis_name="s")

    @functools.partial(
        pl.kernel,
        out_type=jax.ShapeDtypeStruct((NP, D), jnp.float32),
        mesh=mesh,
        scratch_types=[
            pltpu.VMEM((_GPW,), jnp.int32),
            pltpu.VMEM((_GCH, D), jnp.float32),
            pltpu.VMEM((_GCH, D), jnp.float32),
            pltpu.SemaphoreType.DMA,
            pltpu.SemaphoreType.DMA,
            pltpu.SemaphoreType.DMA,
            pltpu.SemaphoreType.DMA,
        ],
    )
    def k(x_hbm, rows_hbm, xs_hbm, idx_v, buf0, buf1, g0, g1, w0, w1):
        wid = lax.axis_index("s") * 2 + lax.axis_index("c")
        base = wid * _GPW
        pltpu.sync_copy(rows_hbm.at[pl.ds(base, _GPW)], idx_v)
        bufs = (buf0, buf1)
        gsems = (g0, g1)
        wsems = (w0, w1)
        gh = [None] * _GNC
        wh = [None] * _GNC
        gh[0] = pltpu.async_copy(
            x_hbm.at[idx_v.at[pl.ds(0, _GCH)]], bufs[0], gsems[0])
        for c in range(_GNC):
            p = c % 2
            gh[c].wait()
            wh[c] = pltpu.async_copy(
                bufs[p], xs_hbm.at[pl.ds(base + c * _GCH, _GCH)], wsems[p])
            if c + 1 < _GNC:
                if c - 1 >= 0:
                    wh[c - 1].wait()
                gh[c + 1] = pltpu.async_copy(
                    x_hbm.at[idx_v.at[pl.ds((c + 1) * _GCH, _GCH)]],
                    bufs[(c + 1) % 2], gsems[(c + 1) % 2])
        wh[_GNC - 2].wait()
        wh[_GNC - 1].wait()

    return k(x2d, rows)


_CCH = 16
_CPW = T // NW


def _sc_combine(ys, inv0, inv1):
    mesh = plsc.VectorSubcoreMesh(core_axis_name="c", subcore_axis_name="s")

    @functools.partial(
        pl.kernel,
        out_type=jax.ShapeDtypeStruct((T, D), jnp.float32),
        mesh=mesh,
        scratch_types=[
            pltpu.VMEM((_CCH,), jnp.int32),
            pltpu.VMEM((_CCH,), jnp.int32),
            pltpu.VMEM((_CCH, D), jnp.float32),
            pltpu.VMEM((_CCH, D), jnp.float32),
            pltpu.SemaphoreType.DMA,
        ],
    )
    def k(ys_hbm, i0_hbm, i1_hbm, out_hbm, i0_v, i1_v, buf0, buf1, sem):
        wid = lax.axis_index("s") * 2 + lax.axis_index("c")
        base = wid * _CPW
        for c in range(_CPW // _CCH):
            off = base + c * _CCH
            pltpu.sync_copy(i0_hbm.at[pl.ds(off, _CCH)], i0_v)
            pltpu.sync_copy(i1_hbm.at[pl.ds(off, _CCH)], i1_v)
            pltpu.async_copy(ys_hbm.at[i0_v], buf0, sem).wait()
            pltpu.async_copy(ys_hbm.at[i1_v], buf1, sem).wait()

            def add_row(r, _):
                for l in range(D // 16):
                    sl = pl.ds(l * 16, 16)
                    buf0[r, sl] = buf0[r, sl] + buf1[r, sl]
                return _

            lax.fori_loop(0, _CCH, add_row, 0)
            pltpu.sync_copy(buf0, out_hbm.at[pl.ds(off, _CCH)])

    return k(ys, inv0, inv1)


def kernel(x, context, quality, params):
    x2d = x.reshape(T, D)
    ctx2d = context.reshape(T, D)

    temp = jnp.clip(params['temp'], 0.25, None)
    wr_s = params['Wr'] / temp
    br_s = (params['br'] / temp).reshape(1, E)
    cvec = (params['bc'] + quality[0] * params['Wq'][:, 0]
            + params['bq']).reshape(1, D)

    idx8, gates8, imp_s, z2_s, ent_s = _router(
        x2d, ctx2d, params['Wc'], wr_s,
        params['rn_g'].reshape(1, D), params['rn_b'].reshape(1, D),
        params['cn_g'].reshape(1, D), params['cn_b'].reshape(1, D),
        cvec, br_s)

    idx2 = idx8[:, :K]
    gates = gates8[:, :K]

    eflat = idx2.reshape(-1)
    counts = jnp.bincount(eflat, length=E)
    blocks_e = (counts + BLK - 1) // BLK
    bstart = jnp.concatenate([jnp.zeros(1, blocks_e.dtype),
                              jnp.cumsum(blocks_e)[:-1]])
    pstart = bstart * BLK
    starts = jnp.concatenate([jnp.zeros(1, counts.dtype),
                              jnp.cumsum(counts)[:-1]])
    order = jnp.argsort(eflat)
    ej = eflat[order]
    j = jnp.arange(T * K)
    ppos = (pstart[ej] + (j - starts[ej])).astype(jnp.int32)
    slot_pair = (jnp.arange(NP, dtype=jnp.int32) % (T * K)).at[ppos].set(
        order.astype(jnp.int32))
    valid = jnp.zeros(NP, jnp.float32).at[ppos].set(1.0)
    rows = slot_pair // K
    gate_slot = (valid * gates.reshape(-1)[slot_pair])[:, None]
    barr = jnp.arange(NBLK)
    nused = jnp.sum(blocks_e).astype(jnp.int32)
    bef = (jnp.sum(barr[:, None] >= bstart[None, :], axis=1) - 1)
    bef = jnp.clip(bef, 0, E - 1).astype(jnp.int32)
    block_expert = jnp.where(barr < nused, bef, bef[nused - 1])
    inv = jnp.zeros(T * K, jnp.int32).at[order].set(ppos)
    inv0 = inv[0::K]
    inv1 = inv[1::K]

    xs = _sc_gather(x2d, rows)
    ys = _grouped_ffn(xs, params['W1'].astype(jnp.bfloat16),
                      params['b1'].reshape(E, 1, H),
                      params['W2'].astype(jnp.bfloat16),
                      params['b2'].reshape(E, 1, D),
                      gate_slot, block_expert, nused.reshape(1))
    out2d = _sc_combine(ys, inv0, inv1)

    invT = 1.0 / jnp.float32(T)
    importance = imp_s[0] * invT
    load_balance = jnp.mean((importance - 1.0 / E) ** 2)
    router_z = z2_s[0, 0] * invT
    entropy = -ent_s[0, 0] * invT
    aux = load_balance + 0.001 * router_z - 0.001 * entropy
    return (out2d.reshape(1, T, D), load_balance, router_z, entropy, aux)
